# baseline (speedup 1.0000x reference)
import jax
import jax.numpy as jnp
from jax import lax
from jax.experimental import pallas as pl
from jax.experimental.pallas import tpu as pltpu
from jax.experimental.pallas import tpu_sc as plsc

N = 100000
DIM = 32
ROWS_PAD = 1007616
NW = 32
B_PER_W = 3128
N_PAD = NW * B_PER_W

TCOLS = 8192
QUART = TCOLS // 4
QUART_LOG2 = QUART.bit_length() - 1
OUT_BLK = TCOLS * DIM // 128
SCRATCH_ROWS = ROWS_PAD * DIM // 128
GRID = -(-ROWS_PAD // TCOLS)


def _transpose_body(x_ref, ei_ref, ea_ref, y_ref, ei_out, ea_out,
                    sem_ei, sem_ea):
    t = pl.program_id(0)

    @pl.when(t == 0)
    def _start():
        pltpu.make_async_copy(ei_ref, ei_out, sem_ei).start()
        pltpu.make_async_copy(ea_ref, ea_out, sem_ea).start()

    x = x_ref[...]
    x4 = jnp.concatenate(
        [x[:, 0:QUART], x[:, QUART:2 * QUART], x[:, 2 * QUART:3 * QUART],
         x[:, 3 * QUART:4 * QUART]], axis=0)
    y_ref[...] = jnp.transpose(x4, (1, 0))

    @pl.when(t == GRID - 1)
    def _wait():
        pltpu.make_async_copy(ei_ref, ei_out, sem_ei).wait()
        pltpu.make_async_copy(ea_ref, ea_out, sem_ea).wait()


def _detile(table_t, edge_index, edge_attr_t):
    return pl.pallas_call(
        _transpose_body,
        grid=(GRID,),
        in_specs=[
            pl.BlockSpec((DIM, TCOLS), lambda t: (0, t)),
            pl.BlockSpec(memory_space=pltpu.HBM),
            pl.BlockSpec(memory_space=pltpu.HBM),
        ],
        out_specs=[
            pl.BlockSpec((OUT_BLK, 128), lambda t: (t, 0)),
            pl.BlockSpec(memory_space=pltpu.HBM),
            pl.BlockSpec(memory_space=pltpu.HBM),
        ],
        out_shape=[
            jax.ShapeDtypeStruct((SCRATCH_ROWS, 128), jnp.float32),
            jax.ShapeDtypeStruct(edge_index.shape, edge_index.dtype),
            jax.ShapeDtypeStruct(edge_attr_t.shape, edge_attr_t.dtype),
        ],
        scratch_shapes=[pltpu.SemaphoreType.DMA, pltpu.SemaphoreType.DMA],
    )(table_t, edge_index, edge_attr_t)


def _gather_body(table_hbm, idx_hbm, out_hbm, idx_v, rows_v, sem):
    wid = lax.axis_index("s") * 2 + lax.axis_index("c")
    base = wid * B_PER_W
    pltpu.sync_copy(idx_hbm.at[pl.ds(base, B_PER_W)], idx_v)
    pltpu.async_copy(table_hbm.at[idx_v], rows_v, sem).wait()
    pltpu.sync_copy(rows_v, out_hbm.at[pl.ds(base, B_PER_W)])


def _gather(table_rows, idx_pad):
    mesh = plsc.VectorSubcoreMesh(core_axis_name="c", subcore_axis_name="s")
    f = pl.kernel(
        _gather_body,
        out_type=jax.ShapeDtypeStruct((N_PAD, DIM), jnp.float32),
        mesh=mesh,
        scratch_types=[
            pltpu.VMEM((B_PER_W,), jnp.int32),
            pltpu.VMEM((B_PER_W, DIM), jnp.float32),
            pltpu.SemaphoreType.DMA,
        ],
        compiler_params=pltpu.CompilerParams(use_tc_tiling_on_sc=False),
    )
    return f(table_rows, idx_pad)


def kernel(x, edge_index, edge_attr, batch, depth, ptr, table):
    table_t = table.T
    scratch, ei_out, ea_out_t = _detile(table_t, edge_index, edge_attr.T)
    table_rows = scratch.reshape(ROWS_PAD, DIM)
    idx = x.reshape(-1)
    u = idx & (TCOLS - 1)
    gidx = (idx - u) + ((u & (QUART - 1)) << 2) + (u >> QUART_LOG2)
    gidx_pad = jnp.pad(gidx, (0, N_PAD - N))
    out = _gather(table_rows, gidx_pad)
    return (out[:N], ei_out, ea_out_t.T, batch, depth, ptr)


# speedup vs baseline: 12.1375x; 12.1375x over previous
import jax
import jax.numpy as jnp
from jax import lax
from jax.experimental import pallas as pl
from jax.experimental.pallas import tpu as pltpu
from jax.experimental.pallas import tpu_sc as plsc

N = 100000
DIM = 32
ROWS_PAD = 1007616
NW = 32

TCOLS = 8192
QUART = TCOLS // 4
QUART_LOG2 = QUART.bit_length() - 1
OUT_BLK = TCOLS * DIM // 128
SCRATCH_ROWS = ROWS_PAD * DIM // 128
GRID = -(-ROWS_PAD // TCOLS)

OCOLS = 2048
OQ = OCOLS // 4
N_PAD = 100352
B_PER_W = N_PAD // NW
OGRID = N_PAD // OCOLS
N_MINOR = 100096


def _transpose_body(x_ref, y_ref):
    x = x_ref[...]
    x4 = jnp.concatenate(
        [x[:, 0:QUART], x[:, QUART:2 * QUART], x[:, 2 * QUART:3 * QUART],
         x[:, 3 * QUART:4 * QUART]], axis=0)
    y_ref[...] = jnp.transpose(x4, (1, 0))


def _detile(table_t):
    return pl.pallas_call(
        _transpose_body,
        grid=(GRID,),
        in_specs=[pl.BlockSpec((DIM, TCOLS), lambda t: (0, t))],
        out_specs=pl.BlockSpec((OUT_BLK, 128), lambda t: (t, 0)),
        out_shape=jax.ShapeDtypeStruct((SCRATCH_ROWS, 128), jnp.float32),
    )(table_t)


def _back_body(g_ref, o_ref):
    z = jnp.transpose(g_ref[...], (1, 0))
    o_ref[...] = jnp.concatenate(
        [z[0:DIM], z[DIM:2 * DIM], z[2 * DIM:3 * DIM], z[3 * DIM:4 * DIM]],
        axis=1)


def _back_transpose(g_flat):
    return pl.pallas_call(
        _back_body,
        grid=(OGRID,),
        in_specs=[pl.BlockSpec((OQ, 128), lambda t: (t, 0))],
        out_specs=pl.BlockSpec((DIM, OCOLS), lambda t: (0, t)),
        out_shape=jax.ShapeDtypeStruct((DIM, N_MINOR), jnp.float32),
    )(g_flat)


def _gather_body(table_hbm, idx_hbm, out_hbm, idx_v, rows_v, sem):
    wid = lax.axis_index("s") * 2 + lax.axis_index("c")
    base = wid * B_PER_W
    pltpu.sync_copy(idx_hbm.at[pl.ds(base, B_PER_W)], idx_v)
    pltpu.async_copy(table_hbm.at[idx_v], rows_v, sem).wait()
    pltpu.sync_copy(rows_v, out_hbm.at[pl.ds(base, B_PER_W)])


def _gather(table_rows, idx_pad):
    mesh = plsc.VectorSubcoreMesh(core_axis_name="c", subcore_axis_name="s")
    f = pl.kernel(
        _gather_body,
        out_type=jax.ShapeDtypeStruct((N_PAD, DIM), jnp.float32),
        mesh=mesh,
        scratch_types=[
            pltpu.VMEM((B_PER_W,), jnp.int32),
            pltpu.VMEM((B_PER_W, DIM), jnp.float32),
            pltpu.SemaphoreType.DMA,
        ],
        compiler_params=pltpu.CompilerParams(use_tc_tiling_on_sc=False),
    )
    return f(table_rows, idx_pad)


def kernel(x, edge_index, edge_attr, batch, depth, ptr, table):
    table_t = table.T
    scratch = _detile(table_t)
    table_rows = scratch.reshape(ROWS_PAD, DIM)
    idx = x.reshape(-1)
    u = idx & (TCOLS - 1)
    gidx = (idx - u) + ((u & (QUART - 1)) << 2) + (u >> QUART_LOG2)
    gp = jnp.pad(gidx, (0, N_PAD - N))
    gidx2 = gp.reshape(OGRID, 4, OQ).transpose(0, 2, 1).reshape(-1)
    g_rows = _gather(table_rows, gidx2)
    out_t = _back_transpose(g_rows.reshape(N_PAD * DIM // 128, 128))
    return (out_t.T[:N], edge_index, edge_attr, batch, depth, ptr)
